# baseline (speedup 1.0000x reference)
import jax
import jax.numpy as jnp
from jax.experimental import pallas as pl
from jax.experimental.pallas import tpu as pltpu

D_MODEL = 1024
E = 64
B = 64
KS = 4
EG = 16
KCH = D_MODEL // KS


def _moe_kernel(x_ref, gw_ref, gb_ref, w_ref, o_ref, gs_ref):
    i = pl.program_id(0)
    j = pl.program_id(1)

    @pl.when((i == 0) & (j == 0))
    def _():
        x = x_ref[...]
        gw = gw_ref[...]
        logits = jax.lax.dot_general(
            x, gw, (((1,), (1,)), ((), ())), preferred_element_type=jnp.float32
        ) + gb_ref[...]
        z = logits - jnp.max(logits, axis=1, keepdims=True)
        ez = jnp.exp(z)
        g = ez / jnp.sum(ez, axis=1, keepdims=True)
        m1 = jnp.max(g, axis=1, keepdims=True)
        g_wo_top1 = jnp.where(g == m1, -1.0, g)
        m2 = jnp.max(g_wo_top1, axis=1, keepdims=True)
        sel = (g >= m2).astype(jnp.float32)
        active = jnp.max(sel, axis=0, keepdims=True)
        gs_ref[...] = g * active

    xk = x_ref[:, pl.ds(j * KCH, KCH)]
    part = jnp.dot(xk, w_ref[...], preferred_element_type=jnp.float32)
    iota0 = jax.lax.broadcasted_iota(jnp.int32, (E, EG), 0)
    iota1 = jax.lax.broadcasted_iota(jnp.int32, (E, EG), 1)
    onehot = (iota0 == EG * i + iota1).astype(jnp.float32)
    cols = jnp.dot(gs_ref[...], onehot, preferred_element_type=jnp.float32)
    contrib = part[:, :D_MODEL] * cols[:, 0:1]
    for k in range(1, EG):
        contrib += part[:, k * D_MODEL:(k + 1) * D_MODEL] * cols[:, k:k + 1]
    o_ref[...] = jnp.where((i == 0) & (j == 0), contrib, o_ref[...] + contrib)


def kernel(x, weight, gate_w, gate_b):
    gb2 = gate_b.reshape(1, E)
    out = pl.pallas_call(
        _moe_kernel,
        grid=(E // EG, KS),
        in_specs=[
            pl.BlockSpec((B, D_MODEL), lambda i, j: (0, 0)),
            pl.BlockSpec((E, D_MODEL), lambda i, j: (0, 0)),
            pl.BlockSpec((1, E), lambda i, j: (0, 0)),
            pl.BlockSpec((KCH, EG * D_MODEL), lambda i, j: (j, i)),
        ],
        out_specs=pl.BlockSpec((B, D_MODEL), lambda i, j: (0, 0)),
        out_shape=jax.ShapeDtypeStruct((B, D_MODEL), jnp.float32),
        scratch_shapes=[pltpu.VMEM((B, E), jnp.float32)],
        compiler_params=pltpu.CompilerParams(
            dimension_semantics=("arbitrary", "arbitrary"),
            vmem_limit_bytes=100 * 1024 * 1024,
        ),
    )(x, gate_w, gb2, weight)
    return out


# speedup vs baseline: 1.0003x; 1.0003x over previous
import jax
import jax.numpy as jnp
from jax.experimental import pallas as pl
from jax.experimental.pallas import tpu as pltpu

D_MODEL = 1024
E = 64
B = 64
KS = 8
EG = 16
KCH = D_MODEL // KS


def _moe_kernel(x_ref, gw_ref, gb_ref, w_ref, o_ref, gs_ref):
    i = pl.program_id(0)
    j = pl.program_id(1)

    @pl.when((i == 0) & (j == 0))
    def _():
        x = x_ref[...]
        gw = gw_ref[...]
        logits = jax.lax.dot_general(
            x, gw, (((1,), (1,)), ((), ())), preferred_element_type=jnp.float32
        ) + gb_ref[...]
        z = logits - jnp.max(logits, axis=1, keepdims=True)
        ez = jnp.exp(z)
        g = ez / jnp.sum(ez, axis=1, keepdims=True)
        m1 = jnp.max(g, axis=1, keepdims=True)
        g_wo_top1 = jnp.where(g == m1, -1.0, g)
        m2 = jnp.max(g_wo_top1, axis=1, keepdims=True)
        sel = (g >= m2).astype(jnp.float32)
        active = jnp.max(sel, axis=0, keepdims=True)
        gs_ref[...] = g * active

    xk = x_ref[:, pl.ds(j * KCH, KCH)]
    part = jnp.dot(xk, w_ref[...], preferred_element_type=jnp.float32)
    iota0 = jax.lax.broadcasted_iota(jnp.int32, (E, EG), 0)
    iota1 = jax.lax.broadcasted_iota(jnp.int32, (E, EG), 1)
    onehot = (iota0 == EG * i + iota1).astype(jnp.float32)
    cols = jnp.dot(gs_ref[...], onehot, preferred_element_type=jnp.float32)
    contrib = part[:, :D_MODEL] * cols[:, 0:1]
    for k in range(1, EG):
        contrib += part[:, k * D_MODEL:(k + 1) * D_MODEL] * cols[:, k:k + 1]
    o_ref[...] = jnp.where((i == 0) & (j == 0), contrib, o_ref[...] + contrib)


def kernel(x, weight, gate_w, gate_b):
    gb2 = gate_b.reshape(1, E)
    out = pl.pallas_call(
        _moe_kernel,
        grid=(E // EG, KS),
        in_specs=[
            pl.BlockSpec((B, D_MODEL), lambda i, j: (0, 0)),
            pl.BlockSpec((E, D_MODEL), lambda i, j: (0, 0)),
            pl.BlockSpec((1, E), lambda i, j: (0, 0)),
            pl.BlockSpec((KCH, EG * D_MODEL), lambda i, j: (j, i)),
        ],
        out_specs=pl.BlockSpec((B, D_MODEL), lambda i, j: (0, 0)),
        out_shape=jax.ShapeDtypeStruct((B, D_MODEL), jnp.float32),
        scratch_shapes=[pltpu.VMEM((B, E), jnp.float32)],
        compiler_params=pltpu.CompilerParams(
            dimension_semantics=("arbitrary", "arbitrary"),
            vmem_limit_bytes=100 * 1024 * 1024,
        ),
    )(x, gate_w, gb2, weight)
    return out


# speedup vs baseline: 1.0142x; 1.0139x over previous
import jax
import jax.numpy as jnp
from jax.experimental import pallas as pl
from jax.experimental.pallas import tpu as pltpu

D_MODEL = 1024
E = 64
B = 64
KS = 2
EG = 8
KCH = D_MODEL // KS


def _moe_kernel(x_ref, gw_ref, gb_ref, w_ref, o_ref, gs_ref):
    i = pl.program_id(0)
    j = pl.program_id(1)

    @pl.when((i == 0) & (j == 0))
    def _():
        x = x_ref[...]
        gw = gw_ref[...]
        logits = jax.lax.dot_general(
            x, gw, (((1,), (1,)), ((), ())), preferred_element_type=jnp.float32
        ) + gb_ref[...]
        z = logits - jnp.max(logits, axis=1, keepdims=True)
        ez = jnp.exp(z)
        g = ez / jnp.sum(ez, axis=1, keepdims=True)
        m1 = jnp.max(g, axis=1, keepdims=True)
        g_wo_top1 = jnp.where(g == m1, -1.0, g)
        m2 = jnp.max(g_wo_top1, axis=1, keepdims=True)
        sel = (g >= m2).astype(jnp.float32)
        active = jnp.max(sel, axis=0, keepdims=True)
        gs_ref[...] = g * active

    xk = x_ref[:, pl.ds(j * KCH, KCH)]
    part = jnp.dot(xk, w_ref[...], preferred_element_type=jnp.float32)
    iota0 = jax.lax.broadcasted_iota(jnp.int32, (E, EG), 0)
    iota1 = jax.lax.broadcasted_iota(jnp.int32, (E, EG), 1)
    onehot = (iota0 == EG * i + iota1).astype(jnp.float32)
    cols = jnp.dot(gs_ref[...], onehot, preferred_element_type=jnp.float32)
    contrib = part[:, :D_MODEL] * cols[:, 0:1]
    for k in range(1, EG):
        contrib += part[:, k * D_MODEL:(k + 1) * D_MODEL] * cols[:, k:k + 1]
    o_ref[...] = jnp.where((i == 0) & (j == 0), contrib, o_ref[...] + contrib)


def kernel(x, weight, gate_w, gate_b):
    gb2 = gate_b.reshape(1, E)
    out = pl.pallas_call(
        _moe_kernel,
        grid=(E // EG, KS),
        in_specs=[
            pl.BlockSpec((B, D_MODEL), lambda i, j: (0, 0)),
            pl.BlockSpec((E, D_MODEL), lambda i, j: (0, 0)),
            pl.BlockSpec((1, E), lambda i, j: (0, 0)),
            pl.BlockSpec((KCH, EG * D_MODEL), lambda i, j: (j, i)),
        ],
        out_specs=pl.BlockSpec((B, D_MODEL), lambda i, j: (0, 0)),
        out_shape=jax.ShapeDtypeStruct((B, D_MODEL), jnp.float32),
        scratch_shapes=[pltpu.VMEM((B, E), jnp.float32)],
        compiler_params=pltpu.CompilerParams(
            dimension_semantics=("arbitrary", "arbitrary"),
            vmem_limit_bytes=100 * 1024 * 1024,
        ),
    )(x, gate_w, gb2, weight)
    return out
